# baseline (speedup 1.0000x reference)
import jax
import jax.numpy as jnp
from jax import lax
from jax.experimental import pallas as pl
from jax.experimental.pallas import tpu as pltpu
from jax.experimental.pallas import tpu_sc as plsc

BATCH = 16384
D = 64
WIDE = 2 * D
NC = 2
NS = 16
NW = NC * NS
NROWS = 100000
CW = 256
NCHUNK = NROWS // CW
TAIL0 = NCHUNK * CW
NTAIL = NROWS - TAIL0
BASE_CH = NCHUNK // NW
EXTRA = NCHUNK - BASE_CH * NW
MAXCH = BASE_CH + 1


def _popcnt(m):
    return plsc.all_reduce_population_count(m)[0]


def _sc_body(user_h, item_h, guT, giT, muT, miT, tail_u, tail_i,
             u_out, i_out,
             sidx, mlist, cg0, cm0, cg1, cm1, stag, posref, rrbuf,
             sema, semb, semc):
    w = lax.axis_index("s") * NC + lax.axis_index("c")
    lo_chunk = BASE_CH * w + jnp.minimum(w, EXTRA)
    ncz = BASE_CH + (w < EXTRA).astype(jnp.int32)
    lo_col = lo_chunk * CW
    is_last = w == NW - 1
    hi_col = jnp.where(is_last, NROWS, (lo_chunk + ncz) * CW)
    iota = lax.iota(jnp.int32, 16)
    bufs = ((cg0, cm0, sema), (cg1, cm1, semb))

    def side(idx_h, tbl_g, tbl_m, tail_t, out):
        pltpu.sync_copy(idx_h, sidx)

        def scan_body(v4, cnt):
            for s in range(4):
                lanes = v4 * 64 + s * 16 + iota
                r = plsc.load_gather(sidx, [lanes])
                m = (r >= lo_col) & (r < hi_col)
                e = (r << 14) | lanes
                plsc.store_compressed(mlist.at[pl.ds(cnt, 16)], e, mask=m)
                cnt = cnt + _popcnt(m)
            return cnt

        n = lax.fori_loop(0, BATCH // 64, scan_body, 0)

        def subfilter(c):
            def body(v4, cnt):
                for s in range(4):
                    lanes = v4 * 64 + s * 16 + iota
                    e = plsc.load_gather(mlist, [jnp.minimum(lanes, n - 1)])
                    m = (lanes < n) & ((e >> 22) == c)
                    plsc.store_compressed(sidx.at[pl.ds(cnt, 16)], e, mask=m)
                    cnt = cnt + _popcnt(m)
                return cnt

            return lax.fori_loop(0, (n + 63) // 64, body, 0)

        def start_dmas(c_rel, par):
            c = lo_chunk + c_rel
            cg, cm, sem = bufs[par]
            for k in range(2):
                col = pl.multiple_of((c * 2 + k) * 128, 128)
                dst = pl.ds(64 * k, 64)
                pltpu.async_copy(tbl_g.at[:, pl.ds(col, 128)], cg.at[dst, :],
                                 sem)
                pltpu.async_copy(tbl_m.at[:, pl.ds(col, 128)], cm.at[dst, :],
                                 sem)

        def drain(c_rel, par):
            c = lo_chunk + c_rel
            cg, cm, sem = bufs[par]
            for k in range(2):
                col = pl.multiple_of((c * 2 + k) * 128, 128)
                dst = pl.ds(64 * k, 64)
                pltpu.make_async_copy(tbl_g.at[:, pl.ds(col, 128)],
                                      cg.at[dst, :], sem).wait()
                pltpu.make_async_copy(tbl_m.at[:, pl.ds(col, 128)],
                                      cm.at[dst, :], sem).wait()

        def gather_chunk(c_rel, par, g_cnt):
            c = lo_chunk + c_rel
            cg, cm, _ = bufs[par]

            def sub_batch(sb, _):
                base = sb * 128
                rrs = []
                for v in range(8):
                    lanes = base + v * 16 + iota
                    e = plsc.load_gather(sidx,
                                         [jnp.minimum(lanes, g_cnt - 1)])
                    rrs.append((e >> 14) - c * CW)
                    posref[pl.ds(v * 16, 16)] = e & (BATCH - 1)

                def dloop(d, _2):
                    dv = jnp.full((16,), d, jnp.int32)
                    for v in range(8):
                        rows = v * 16 + iota
                        rr = rrs[v]
                        di = dv + ((rr >> 7) << 6)
                        cl = rr & 127
                        g = plsc.load_gather(cg, [di, cl])
                        plsc.store_scatter(stag, [rows, dv], g)
                        m = plsc.load_gather(cm, [di, cl])
                        plsc.store_scatter(stag, [rows, dv + D], m)
                    return 0

                lax.fori_loop(0, D, dloop, 0)
                pltpu.async_copy(stag, out.at[posref], semc).wait()
                return 0

            lax.fori_loop(0, (g_cnt + 127) // 128, sub_batch, 0)

        def half(c_rel, par):
            @pl.when(c_rel + 1 < ncz)
            def _pre():
                start_dmas(c_rel + 1, 1 - par)

            g_cnt = subfilter(lo_chunk + c_rel)
            drain(c_rel, par)
            gather_chunk(c_rel, par, g_cnt)

        start_dmas(0, 0)

        def pair(pr, _):
            c0 = 2 * pr

            @pl.when(c0 < ncz)
            def _a():
                half(c0, 0)

            @pl.when(c0 + 1 < ncz)
            def _b():
                half(c0 + 1, 1)

            return 0

        lax.fori_loop(0, (MAXCH + 1) // 2, pair, 0)

        @pl.when(is_last)
        def _tail():
            g_cnt = subfilter(NCHUNK)

            def sub_batch(sb, _):
                base = sb * 128
                for v in range(8):
                    lanes = base + v * 16 + iota
                    e = plsc.load_gather(sidx, [jnp.minimum(lanes, g_cnt - 1)])
                    rrbuf[pl.ds(v * 16, 16)] = (e >> 14) - TAIL0
                    posref[pl.ds(v * 16, 16)] = e & (BATCH - 1)
                pltpu.async_copy(tail_t.at[rrbuf], stag, semc).wait()
                pltpu.async_copy(stag, out.at[posref], semc).wait()
                return 0

            lax.fori_loop(0, (g_cnt + 127) // 128, sub_batch, 0)

    side(user_h, guT, muT, tail_u, u_out)
    side(item_h, giT, miT, tail_i, i_out)


def _make_sc():
    mesh = plsc.VectorSubcoreMesh(core_axis_name="c", subcore_axis_name="s")
    f32, i32 = jnp.float32, jnp.int32
    row = jax.ShapeDtypeStruct((BATCH, WIDE), f32)
    return pl.kernel(
        _sc_body,
        out_type=[row, row],
        mesh=mesh,
        scratch_types=[
            pltpu.VMEM((BATCH,), i32),
            pltpu.VMEM((BATCH,), i32),
            pltpu.VMEM((2 * D, 128), f32),
            pltpu.VMEM((2 * D, 128), f32),
            pltpu.VMEM((2 * D, 128), f32),
            pltpu.VMEM((2 * D, 128), f32),
            pltpu.VMEM((128, WIDE), f32),
            pltpu.VMEM((128,), i32),
            pltpu.VMEM((128,), i32),
            pltpu.SemaphoreType.DMA,
            pltpu.SemaphoreType.DMA,
            pltpu.SemaphoreType.DMA,
        ],
        compiler_params=pltpu.CompilerParams(use_tc_tiling_on_sc=True,
                                             needs_layout_passes=False),
    )


BLK = 512


def _tc_body(u, i, w0, b0, w1, b1, hw, hb, nw, nb, fused_o, score_o):
    gu = u[:, :D]
    mu = u[:, D:]
    gi = i[:, :D]
    mi = i[:, D:]
    mlp_x = jnp.concatenate([mu, mi], axis=1)
    h = jnp.maximum(jnp.dot(mlp_x, w0[...],
                            preferred_element_type=jnp.float32) + b0[...], 0.0)
    mlp_out = jnp.maximum(jnp.dot(h, w1[...],
                                  preferred_element_type=jnp.float32) + b1[...], 0.0)
    fused_in = jnp.concatenate([0.5 * gu * gi, 0.5 * mlp_out], axis=1)
    fused = jnp.dot(fused_in, hw[...],
                    preferred_element_type=jnp.float32) + hb[...]
    fused_o[...] = fused
    score_o[...] = jnp.sum(fused * nw[...], axis=1) + nb[0, 0]


def _make_tc():
    grid = (BATCH // BLK,)
    blk_in = pl.BlockSpec((BLK, WIDE), lambda i: (i, 0))
    full = lambda shape: pl.BlockSpec(shape, lambda i: (0, 0))
    return pl.pallas_call(
        _tc_body,
        grid=grid,
        in_specs=[
            blk_in, blk_in,
            full((WIDE, WIDE)),
            full((1, WIDE)),
            full((WIDE, D)),
            full((1, D)),
            full((WIDE, D)),
            full((1, D)),
            full((1, D)),
            full((1, 1)),
        ],
        out_specs=[
            pl.BlockSpec((BLK, D), lambda i: (i, 0)),
            pl.BlockSpec((BLK,), lambda i: (i,)),
        ],
        out_shape=[
            jax.ShapeDtypeStruct((BATCH, D), jnp.float32),
            jax.ShapeDtypeStruct((BATCH,), jnp.float32),
        ],
    )


def kernel(user, item, gmf_user_table, gmf_item_table, mlp_user_table,
           mlp_item_table, mlp_W0, mlp_b0, mlp_W1, mlp_b1,
           hidden_W, hidden_b, nmf_W, nmf_b):
    user = user.astype(jnp.int32)
    item = item.astype(jnp.int32)
    guT = gmf_user_table.T
    giT = gmf_item_table.T
    muT = mlp_user_table.T
    miT = mlp_item_table.T
    tail_u = jnp.concatenate([gmf_user_table[TAIL0:], mlp_user_table[TAIL0:]],
                             axis=1)
    tail_i = jnp.concatenate([gmf_item_table[TAIL0:], mlp_item_table[TAIL0:]],
                             axis=1)
    u_rows, i_rows = _make_sc()(user, item, guT, giT, muT, miT, tail_u, tail_i)
    fused, score = _make_tc()(
        u_rows, i_rows,
        mlp_W0, mlp_b0.reshape(1, -1), mlp_W1, mlp_b1.reshape(1, -1),
        hidden_W, hidden_b.reshape(1, -1),
        nmf_W.reshape(1, -1), nmf_b.reshape(1, 1))
    return (score.reshape(BATCH, 1), fused)


# speedup vs baseline: 1.5894x; 1.5894x over previous
import jax
import jax.numpy as jnp
from jax import lax
from jax.experimental import pallas as pl
from jax.experimental.pallas import tpu as pltpu
from jax.experimental.pallas import tpu_sc as plsc

BATCH = 16384
DIM = 64
WIDE = 2 * DIM
NC = 2
NS = 16
NW = NC * NS
PER_W = BATCH // NW
CHUNK = 128
NCH = PER_W // CHUNK


def _sc_gather_body(user_hbm, item_hbm, bu_t, bi_t, u_o, i_o,
                    idx0, idx1, buf0, buf1, sem0, sem1):
    wid = lax.axis_index("s") * NC + lax.axis_index("c")
    base = wid * PER_W
    tasks = [(user_hbm, bu_t, u_o, c) for c in range(NCH)] + \
            [(item_hbm, bi_t, i_o, c) for c in range(NCH)]
    idxs = (idx0, idx1)
    bufs = (buf0, buf1)
    sems = (sem0, sem1)

    def start(t):
        src_idx, tbl, _, c = tasks[t]
        pltpu.sync_copy(src_idx.at[pl.ds(base + c * CHUNK, CHUNK)],
                        idxs[t % 2])
        return pltpu.async_copy(tbl.at[idxs[t % 2]], bufs[t % 2], sems[t % 2])

    cp = start(0)
    for t in range(len(tasks)):
        nxt = None
        if t + 1 < len(tasks):
            nxt = start(t + 1)
        cp.wait()
        _, _, out, c = tasks[t]
        pltpu.sync_copy(bufs[t % 2], out.at[pl.ds(base + c * CHUNK, CHUNK)])
        cp = nxt


def _make_sc_gather():
    mesh = plsc.VectorSubcoreMesh(core_axis_name="c", subcore_axis_name="s")
    row = jax.ShapeDtypeStruct((BATCH, WIDE), jnp.float32)
    return pl.kernel(
        _sc_gather_body,
        out_type=[row, row],
        mesh=mesh,
        scratch_types=[
            pltpu.VMEM((CHUNK,), jnp.int32),
            pltpu.VMEM((CHUNK,), jnp.int32),
            pltpu.VMEM((CHUNK, WIDE), jnp.float32),
            pltpu.VMEM((CHUNK, WIDE), jnp.float32),
            pltpu.SemaphoreType.DMA,
            pltpu.SemaphoreType.DMA,
        ],
        compiler_params=pltpu.CompilerParams(use_tc_tiling_on_sc=True),
    )


BLK = 512


def _tc_dense_body(u, i, w0, b0, w1, b1, hw, hb, nw, nb,
                   fused_o, score_o):
    gu = u[:, :DIM]
    mu = u[:, DIM:]
    gi = i[:, :DIM]
    mi = i[:, DIM:]
    mlp_x = jnp.concatenate([mu, mi], axis=1)
    h = jnp.maximum(jnp.dot(mlp_x, w0[...],
                            preferred_element_type=jnp.float32) + b0[...], 0.0)
    mlp_out = jnp.maximum(jnp.dot(h, w1[...],
                                  preferred_element_type=jnp.float32) + b1[...], 0.0)
    gmf = gu * gi
    fused_in = jnp.concatenate([0.5 * gmf, 0.5 * mlp_out], axis=1)
    fused = jnp.dot(fused_in, hw[...],
                    preferred_element_type=jnp.float32) + hb[...]
    fused_o[...] = fused
    score_o[...] = jnp.sum(fused * nw[...], axis=1) + nb[0, 0]


def _make_tc_dense():
    grid = (BATCH // BLK,)
    blk_in = pl.BlockSpec((BLK, WIDE), lambda i: (i, 0))
    full = lambda shape: pl.BlockSpec(shape, lambda i: (0, 0))
    return pl.pallas_call(
        _tc_dense_body,
        grid=grid,
        in_specs=[
            blk_in, blk_in,
            full((WIDE, WIDE)),
            full((1, WIDE)),
            full((WIDE, DIM)),
            full((1, DIM)),
            full((WIDE, DIM)),
            full((1, DIM)),
            full((1, DIM)),
            full((1, 1)),
        ],
        out_specs=[
            pl.BlockSpec((BLK, DIM), lambda i: (i, 0)),
            pl.BlockSpec((BLK,), lambda i: (i,)),
        ],
        out_shape=[
            jax.ShapeDtypeStruct((BATCH, DIM), jnp.float32),
            jax.ShapeDtypeStruct((BATCH,), jnp.float32),
        ],
    )


def kernel(user, item, gmf_user_table, gmf_item_table, mlp_user_table,
           mlp_item_table, mlp_W0, mlp_b0, mlp_W1, mlp_b1,
           hidden_W, hidden_b, nmf_W, nmf_b):
    user = user.astype(jnp.int32)
    item = item.astype(jnp.int32)
    big_u = jnp.concatenate([gmf_user_table, mlp_user_table], axis=1)
    big_i = jnp.concatenate([gmf_item_table, mlp_item_table], axis=1)
    u_rows, i_rows = _make_sc_gather()(user, item, big_u, big_i)
    fused, score = _make_tc_dense()(
        u_rows, i_rows,
        mlp_W0, mlp_b0.reshape(1, -1), mlp_W1, mlp_b1.reshape(1, -1),
        hidden_W, hidden_b.reshape(1, -1),
        nmf_W.reshape(1, -1), nmf_b.reshape(1, 1))
    return (score.reshape(BATCH, 1), fused)


# speedup vs baseline: 1.7113x; 1.0767x over previous
import jax
import jax.numpy as jnp
from jax import lax
from jax.experimental import pallas as pl
from jax.experimental.pallas import tpu as pltpu
from jax.experimental.pallas import tpu_sc as plsc

BATCH = 16384
DIM = 64
WIDE = 2 * DIM
NC = 2
NS = 16
NW = NC * NS
PER_W = BATCH // NW
CHUNK = 128
NCH = PER_W // CHUNK


def _sc_gather_body(user_hbm, bu_t, u_o,
                    idx0, idx1, buf0, buf1, sem0, sem1):
    wid = lax.axis_index("s") * NC + lax.axis_index("c")
    base = wid * PER_W
    tasks = [(user_hbm, bu_t, u_o, c) for c in range(NCH)]
    idxs = (idx0, idx1)
    bufs = (buf0, buf1)
    sems = (sem0, sem1)

    def start(t):
        src_idx, tbl, _, c = tasks[t]
        pltpu.sync_copy(src_idx.at[pl.ds(base + c * CHUNK, CHUNK)],
                        idxs[t % 2])
        return pltpu.async_copy(tbl.at[idxs[t % 2]], bufs[t % 2], sems[t % 2])

    cp = start(0)
    for t in range(len(tasks)):
        nxt = None
        if t + 1 < len(tasks):
            nxt = start(t + 1)
        cp.wait()
        _, _, out, c = tasks[t]
        pltpu.sync_copy(bufs[t % 2], out.at[pl.ds(base + c * CHUNK, CHUNK)])
        cp = nxt


def _make_sc_gather():
    mesh = plsc.VectorSubcoreMesh(core_axis_name="c", subcore_axis_name="s")
    row = jax.ShapeDtypeStruct((BATCH, WIDE), jnp.float32)
    return pl.kernel(
        _sc_gather_body,
        out_type=row,
        mesh=mesh,
        scratch_types=[
            pltpu.VMEM((CHUNK,), jnp.int32),
            pltpu.VMEM((CHUNK,), jnp.int32),
            pltpu.VMEM((CHUNK, WIDE), jnp.float32),
            pltpu.VMEM((CHUNK, WIDE), jnp.float32),
            pltpu.SemaphoreType.DMA,
            pltpu.SemaphoreType.DMA,
        ],
        compiler_params=pltpu.CompilerParams(use_tc_tiling_on_sc=True),
    )


BLK = 1024


def _tc_dense_body(u, i, w0, b0, w1, b1, hw, hb, nw, nb,
                   fused_o, score_o):
    gu = u[:, :DIM]
    mu = u[:, DIM:]
    gi = i[:, :DIM]
    mi = i[:, DIM:]
    mlp_x = jnp.concatenate([mu, mi], axis=1)
    h = jnp.maximum(jnp.dot(mlp_x, w0[...],
                            preferred_element_type=jnp.float32) + b0[...], 0.0)
    mlp_out = jnp.maximum(jnp.dot(h, w1[...],
                                  preferred_element_type=jnp.float32) + b1[...], 0.0)
    gmf = gu * gi
    fused_in = jnp.concatenate([0.5 * gmf, 0.5 * mlp_out], axis=1)
    fused = jnp.dot(fused_in, hw[...],
                    preferred_element_type=jnp.float32) + hb[...]
    fused_o[...] = fused
    score_o[...] = jnp.sum(fused * nw[...], axis=1) + nb[0, 0]


def _make_tc_dense():
    grid = (BATCH // BLK,)
    blk_in = pl.BlockSpec((BLK, WIDE), lambda i: (i, 0))
    full = lambda shape: pl.BlockSpec(shape, lambda i: (0, 0))
    return pl.pallas_call(
        _tc_dense_body,
        grid=grid,
        in_specs=[
            blk_in, blk_in,
            full((WIDE, WIDE)),
            full((1, WIDE)),
            full((WIDE, DIM)),
            full((1, DIM)),
            full((WIDE, DIM)),
            full((1, DIM)),
            full((1, DIM)),
            full((1, 1)),
        ],
        out_specs=[
            pl.BlockSpec((BLK, DIM), lambda i: (i, 0)),
            pl.BlockSpec((BLK,), lambda i: (i,)),
        ],
        out_shape=[
            jax.ShapeDtypeStruct((BATCH, DIM), jnp.float32),
            jax.ShapeDtypeStruct((BATCH,), jnp.float32),
        ],
    )


def kernel(user, item, gmf_user_table, gmf_item_table, mlp_user_table,
           mlp_item_table, mlp_W0, mlp_b0, mlp_W1, mlp_b1,
           hidden_W, hidden_b, nmf_W, nmf_b):
    user = user.astype(jnp.int32)
    item = item.astype(jnp.int32)
    big_u = jnp.concatenate([gmf_user_table, mlp_user_table], axis=1)
    u_rows = _make_sc_gather()(user, big_u)
    big_i = jnp.concatenate([gmf_item_table, mlp_item_table], axis=1)
    i_rows = _make_sc_gather()(item, big_i)
    fused, score = _make_tc_dense()(
        u_rows, i_rows,
        mlp_W0, mlp_b0.reshape(1, -1), mlp_W1, mlp_b1.reshape(1, -1),
        hidden_W, hidden_b.reshape(1, -1),
        nmf_W.reshape(1, -1), nmf_b.reshape(1, 1))
    return (score.reshape(BATCH, 1), fused)


# speedup vs baseline: 1.7171x; 1.0034x over previous
import jax
import jax.numpy as jnp
from jax import lax
from jax.experimental import pallas as pl
from jax.experimental.pallas import tpu as pltpu
from jax.experimental.pallas import tpu_sc as plsc

BATCH = 16384
DIM = 64
WIDE = 2 * DIM
NC = 2
NS = 16
NW = NC * NS
PER_W = BATCH // NW
CHUNK = 128
NCH = PER_W // CHUNK


def _sc_gather_body(user_hbm, bu_t, u_o,
                    idx0, idx1, buf0, buf1, sem0, sem1):
    wid = lax.axis_index("s") * NC + lax.axis_index("c")
    base = wid * PER_W
    tasks = [(user_hbm, bu_t, u_o, c) for c in range(NCH)]
    idxs = (idx0, idx1)
    bufs = (buf0, buf1)
    sems = (sem0, sem1)

    def start(t):
        src_idx, tbl, _, c = tasks[t]
        pltpu.sync_copy(src_idx.at[pl.ds(base + c * CHUNK, CHUNK)],
                        idxs[t % 2])
        return pltpu.async_copy(tbl.at[idxs[t % 2]], bufs[t % 2], sems[t % 2])

    cp = start(0)
    for t in range(len(tasks)):
        nxt = None
        if t + 1 < len(tasks):
            nxt = start(t + 1)
        cp.wait()
        _, _, out, c = tasks[t]
        pltpu.sync_copy(bufs[t % 2], out.at[pl.ds(base + c * CHUNK, CHUNK)])
        cp = nxt


def _make_sc_gather():
    mesh = plsc.VectorSubcoreMesh(core_axis_name="c", subcore_axis_name="s")
    row = jax.ShapeDtypeStruct((BATCH, WIDE), jnp.float32)
    return pl.kernel(
        _sc_gather_body,
        out_type=row,
        mesh=mesh,
        scratch_types=[
            pltpu.VMEM((CHUNK,), jnp.int32),
            pltpu.VMEM((CHUNK,), jnp.int32),
            pltpu.VMEM((CHUNK, WIDE), jnp.float32),
            pltpu.VMEM((CHUNK, WIDE), jnp.float32),
            pltpu.SemaphoreType.DMA,
            pltpu.SemaphoreType.DMA,
        ],
        compiler_params=pltpu.CompilerParams(use_tc_tiling_on_sc=True),
    )


BLK = 1024


def _tc_dense_body(u, i, w0, b0, w1, b1, hw, hb, nw, nb,
                   fused_o, score_o):
    gu = u[:, :DIM]
    mu = u[:, DIM:]
    gi = i[:, :DIM]
    mi = i[:, DIM:]
    mlp_x = jnp.concatenate([mu, mi], axis=1)
    h = jnp.maximum(jnp.dot(mlp_x, w0[...],
                            preferred_element_type=jnp.float32) + b0[...], 0.0)
    mlp_out = jnp.maximum(jnp.dot(h, w1[...],
                                  preferred_element_type=jnp.float32) + b1[...], 0.0)
    gmf = gu * gi
    fused_in = jnp.concatenate([0.5 * gmf, 0.5 * mlp_out], axis=1)
    fused = jnp.dot(fused_in, hw[...],
                    preferred_element_type=jnp.float32) + hb[...]
    fused_o[...] = fused
    score_o[...] = jnp.sum(fused * nw[...], axis=1) + nb[0, 0]


def _make_tc_dense():
    grid = (BATCH // BLK,)
    blk_in = pl.BlockSpec((BLK, WIDE), lambda i: (i, 0))
    full = lambda shape: pl.BlockSpec(shape, lambda i: (0, 0))
    return pl.pallas_call(
        _tc_dense_body,
        grid=grid,
        in_specs=[
            blk_in, blk_in,
            full((WIDE, WIDE)),
            full((1, WIDE)),
            full((WIDE, DIM)),
            full((1, DIM)),
            full((WIDE, DIM)),
            full((1, DIM)),
            full((1, DIM)),
            full((1, 1)),
        ],
        out_specs=[
            pl.BlockSpec((BLK, DIM), lambda i: (i, 0)),
            pl.BlockSpec((BLK,), lambda i: (i,)),
        ],
        out_shape=[
            jax.ShapeDtypeStruct((BATCH, DIM), jnp.float32),
            jax.ShapeDtypeStruct((BATCH,), jnp.float32),
        ],
    )


def kernel(user, item, gmf_user_table, gmf_item_table, mlp_user_table,
           mlp_item_table, mlp_W0, mlp_b0, mlp_W1, mlp_b1,
           hidden_W, hidden_b, nmf_W, nmf_b):
    user = user.astype(jnp.int32)
    item = item.astype(jnp.int32)
    big_u = jnp.concatenate([gmf_user_table.T, mlp_user_table.T], axis=0).T
    u_rows = _make_sc_gather()(user, big_u)
    big_i = jnp.concatenate([gmf_item_table.T, mlp_item_table.T], axis=0).T
    i_rows = _make_sc_gather()(item, big_i)
    fused, score = _make_tc_dense()(
        u_rows, i_rows,
        mlp_W0, mlp_b0.reshape(1, -1), mlp_W1, mlp_b1.reshape(1, -1),
        hidden_W, hidden_b.reshape(1, -1),
        nmf_W.reshape(1, -1), nmf_b.reshape(1, 1))
    return (score.reshape(BATCH, 1), fused)
